# baseline (speedup 1.0000x reference)
import functools

import jax
import jax.numpy as jnp
from jax import lax
from jax.experimental import pallas as pl
from jax.experimental.pallas import tpu as pltpu
from jax.experimental.pallas import tpu_sc as plsc

N = 320000
D = 128
B_SEG = 10000

NHALF = 2
NH = N // NHALF

NC = 2
NS = 16
NW = NC * NS

ROWS_PER_W = NH // NW
CHUNK = 40
NCHUNK = ROWS_PER_W // CHUNK
NPAIR = (NCHUNK - 1) // 2
B_PAD = 10240
ZROWS = B_PAD // NS

RB = 8000
STEPS_H = NH // RB
RBC = 2000


def _gate_body(x_ref, r_ref, w12_ref, b_ref, y_ref):
    s = (jnp.dot(x_ref[...], w12_ref[..., 0:1],
                 preferred_element_type=jnp.float32)
         + jnp.dot(r_ref[...], w12_ref[..., 1:2],
                   preferred_element_type=jnp.float32)
         + b_ref[0, 0])
    y_ref[...] = jnp.tanh(s) * x_ref[...]


def _gate_half(h, x, ref, w12, b):
    return pl.pallas_call(
        _gate_body,
        grid=(STEPS_H,),
        in_specs=[
            pl.BlockSpec((RB, D), lambda i: (i + h * STEPS_H, 0)),
            pl.BlockSpec((RB, D), lambda i: (i + h * STEPS_H, 0)),
            pl.BlockSpec((D, 2), lambda i: (0, 0)),
            pl.BlockSpec(memory_space=pltpu.SMEM),
        ],
        out_specs=pl.BlockSpec((RB, D), lambda i: (i, 0)),
        out_shape=jax.ShapeDtypeStruct((NH, D), jnp.float32),
    )(x, ref, w12, b)


@functools.cache
def _make_sc_segsum():
    mesh = plsc.VectorSubcoreMesh(
        core_axis_name="c", subcore_axis_name="s",
        num_cores=NC, num_subcores=NS)
    return functools.partial(
        pl.kernel,
        out_type=jax.ShapeDtypeStruct((NC, B_PAD, D), jnp.float32),
        mesh=mesh,
        scratch_types=[
            pltpu.VMEM((NCHUNK, CHUNK), jnp.int32),
            pltpu.VMEM((CHUNK, D), jnp.float32),
            pltpu.VMEM((CHUNK, D), jnp.float32),
            pltpu.VMEM_SHARED((B_PAD, D), jnp.float32),
            pltpu.SemaphoreType.DMA,
            pltpu.SemaphoreType.DMA,
        ],
    )(_sc_segsum_body)


def _sc_segsum_body(y_hbm, idx3_hbm, zeros_hbm, out_hbm,
                    idx_v, rows0, rows1, acc_sh, semg0, semg1):
    cid = lax.axis_index("c")
    sid = lax.axis_index("s")
    wid = cid * NS + sid

    zoff = pl.multiple_of(sid * ZROWS, 8)
    pltpu.sync_copy(zeros_hbm.at[pl.ds(zoff, ZROWS)],
                    acc_sh.at[pl.ds(zoff, ZROWS)])
    pltpu.sync_copy(idx3_hbm.at[wid], idx_v)
    plsc.subcore_barrier()

    base = wid * ROWS_PER_W

    def chunk_slice(c):
        return y_hbm.at[pl.ds(pl.multiple_of(base + c * CHUNK, 8), CHUNK)]

    pltpu.async_copy(chunk_slice(0), rows0, semg0)

    def pair_body(i, carry):
        a = 2 * i
        pltpu.async_copy(chunk_slice(a + 1), rows1, semg1)
        pltpu.make_async_copy(chunk_slice(a), rows0, semg0).wait()
        pltpu.sync_copy(rows0, acc_sh.at[idx_v.at[a]], add=True)
        pltpu.async_copy(chunk_slice(a + 2), rows0, semg0)
        pltpu.make_async_copy(chunk_slice(a + 1), rows1, semg1).wait()
        pltpu.sync_copy(rows1, acc_sh.at[idx_v.at[a + 1]], add=True)
        return carry

    lax.fori_loop(0, NPAIR, pair_body, 0)

    pltpu.make_async_copy(chunk_slice(NCHUNK - 1), rows0, semg0).wait()
    pltpu.sync_copy(rows0, acc_sh.at[idx_v.at[NCHUNK - 1]], add=True)

    plsc.subcore_barrier()
    pltpu.sync_copy(acc_sh.at[pl.ds(zoff, ZROWS)],
                    out_hbm.at[cid, pl.ds(zoff, ZROWS)])


def _combine_body(p0_ref, p1_ref, o_ref):
    o_ref[...] = (p0_ref[0] + p0_ref[1]) + (p1_ref[0] + p1_ref[1])


def _combine(p0, p1):
    return pl.pallas_call(
        _combine_body,
        grid=(B_SEG // RBC,),
        in_specs=[
            pl.BlockSpec((NC, RBC, D), lambda i: (0, i, 0)),
            pl.BlockSpec((NC, RBC, D), lambda i: (0, i, 0)),
        ],
        out_specs=pl.BlockSpec((RBC, D), lambda i: (i, 0)),
        out_shape=jax.ShapeDtypeStruct((B_SEG, D), jnp.float32),
    )(p0, p1)


def kernel(x, ref, index, batch_size, W, b):
    x = x.astype(jnp.float32)
    ref = ref.astype(jnp.float32)
    w12 = W.reshape(2, D).T.astype(jnp.float32)
    b2 = b.reshape(1, 1).astype(jnp.float32)
    idx4 = index.astype(jnp.int32).reshape(NHALF, NW, NCHUNK, CHUNK)
    zeros = jnp.zeros((B_PAD, D), jnp.float32)
    sc_segsum = _make_sc_segsum()
    y0 = _gate_half(0, x, ref, w12, b2)
    y1 = _gate_half(1, x, ref, w12, b2)
    p0 = sc_segsum(y0, idx4[0], zeros)
    p1 = sc_segsum(y1, idx4[1], zeros)
    return _combine(p0, p1)


# speedup vs baseline: 1.0910x; 1.0910x over previous
import functools

import jax
import jax.numpy as jnp
from jax import lax
from jax.experimental import pallas as pl
from jax.experimental.pallas import tpu as pltpu
from jax.experimental.pallas import tpu_sc as plsc

N = 320000
D = 128
B_SEG = 10000

NC = 2
NS = 16
NW = NC * NS

ROWS_PER_W = N // NW
CHUNK = 80
NCHUNK = ROWS_PER_W // CHUNK
NTRI = (NCHUNK - 2) // 3
B_PAD = 10240
ZROWS = B_PAD // NS

RB = 12800
RBC = 2000


def _gate_body(x_ref, r_ref, w12_ref, b_ref, y_ref):
    s = (jnp.dot(x_ref[...], w12_ref[..., 0:1],
                 preferred_element_type=jnp.float32)
         + jnp.dot(r_ref[...], w12_ref[..., 1:2],
                   preferred_element_type=jnp.float32)
         + b_ref[0, 0])
    y_ref[...] = jnp.tanh(s) * x_ref[...]


def _gate(x, ref, w12, b):
    return pl.pallas_call(
        _gate_body,
        grid=(N // RB,),
        in_specs=[
            pl.BlockSpec((RB, D), lambda i: (i, 0)),
            pl.BlockSpec((RB, D), lambda i: (i, 0)),
            pl.BlockSpec((D, 2), lambda i: (0, 0)),
            pl.BlockSpec(memory_space=pltpu.SMEM),
        ],
        out_specs=pl.BlockSpec((RB, D), lambda i: (i, 0)),
        out_shape=jax.ShapeDtypeStruct((N, D), jnp.float32),
    )(x, ref, w12, b)


@functools.cache
def _make_sc_segsum():
    mesh = plsc.VectorSubcoreMesh(
        core_axis_name="c", subcore_axis_name="s",
        num_cores=NC, num_subcores=NS)
    return functools.partial(
        pl.kernel,
        out_type=jax.ShapeDtypeStruct((NC, B_PAD, D), jnp.float32),
        mesh=mesh,
        scratch_types=[
            pltpu.VMEM((NCHUNK, CHUNK), jnp.int32),
            pltpu.VMEM((CHUNK, D), jnp.float32),
            pltpu.VMEM((CHUNK, D), jnp.float32),
            pltpu.VMEM((CHUNK, D), jnp.float32),
            pltpu.VMEM_SHARED((B_PAD, D), jnp.float32),
            pltpu.SemaphoreType.DMA,
            pltpu.SemaphoreType.DMA,
            pltpu.SemaphoreType.DMA,
            pltpu.SemaphoreType.DMA,
            pltpu.SemaphoreType.DMA,
            pltpu.SemaphoreType.DMA,
        ],
    )(_sc_segsum_body)


def _sc_segsum_body(y_hbm, idx3_hbm, zeros_hbm, out_hbm,
                    idx_v, rows0, rows1, rows2, acc_sh,
                    semg0, semg1, semg2, sems0, sems1, sems2):
    cid = lax.axis_index("c")
    sid = lax.axis_index("s")
    wid = cid * NS + sid
    rows = (rows0, rows1, rows2)
    semg = (semg0, semg1, semg2)
    sems = (sems0, sems1, sems2)

    zoff = pl.multiple_of(sid * ZROWS, 8)
    pltpu.sync_copy(zeros_hbm.at[pl.ds(zoff, ZROWS)],
                    acc_sh.at[pl.ds(zoff, ZROWS)])
    pltpu.sync_copy(idx3_hbm.at[wid], idx_v)
    plsc.subcore_barrier()

    base = wid * ROWS_PER_W

    def chunk_slice(c):
        return y_hbm.at[pl.ds(pl.multiple_of(base + c * CHUNK, 8), CHUNK)]

    def wait_gather(c, k):
        pltpu.make_async_copy(chunk_slice(c), rows[k], semg[k]).wait()

    def wait_scat(c, k):
        pltpu.make_async_copy(rows[k], acc_sh.at[idx_v.at[c]], sems[k]).wait()

    pltpu.async_copy(chunk_slice(0), rows0, semg0)

    def tri_body(i, carry):
        c0 = 3 * i
        for k in range(3):
            c = c0 + k
            kn = (k + 1) % 3

            @pl.when((c >= 2) | (k == 2))
            def _():
                wait_scat(c - 2, kn)
            pltpu.async_copy(chunk_slice(c + 1), rows[kn], semg[kn])
            wait_gather(c, k)
            pltpu.async_copy(rows[k], acc_sh.at[idx_v.at[c]], sems[k], add=True)
        return carry

    lax.fori_loop(0, NTRI, tri_body, 0)

    t = NCHUNK - 2
    wait_scat(t - 2, 1)
    pltpu.async_copy(chunk_slice(t + 1), rows1, semg1)
    wait_gather(t, 0)
    pltpu.async_copy(rows0, acc_sh.at[idx_v.at[t]], sems0, add=True)
    wait_scat(t - 1, 2)
    wait_gather(t + 1, 1)
    pltpu.async_copy(rows1, acc_sh.at[idx_v.at[t + 1]], sems1, add=True)
    wait_scat(t, 0)
    wait_scat(t + 1, 1)

    plsc.subcore_barrier()
    pltpu.sync_copy(acc_sh.at[pl.ds(zoff, ZROWS)],
                    out_hbm.at[cid, pl.ds(zoff, ZROWS)])


def _combine_body(p0_ref, p1_ref, o_ref):
    o_ref[...] = p0_ref[...] + p1_ref[...]


def _combine(p0, p1):
    return pl.pallas_call(
        _combine_body,
        grid=(B_SEG // RBC,),
        in_specs=[
            pl.BlockSpec((RBC, D), lambda i: (i, 0)),
            pl.BlockSpec((RBC, D), lambda i: (i, 0)),
        ],
        out_specs=pl.BlockSpec((RBC, D), lambda i: (i, 0)),
        out_shape=jax.ShapeDtypeStruct((B_SEG, D), jnp.float32),
    )(p0, p1)


def kernel(x, ref, index, batch_size, W, b):
    x = x.astype(jnp.float32)
    ref = ref.astype(jnp.float32)
    w12 = W.reshape(2, D).T.astype(jnp.float32)
    b2 = b.reshape(1, 1).astype(jnp.float32)
    y = _gate(x, ref, w12, b2)
    idx3 = index.astype(jnp.int32).reshape(NW, NCHUNK, CHUNK)
    zeros = jnp.zeros((B_PAD, D), jnp.float32)
    partials = _make_sc_segsum()(y, idx3, zeros)
    return _combine(partials[0], partials[1])
